# baseline (speedup 1.0000x reference)
import functools

import jax
import jax.numpy as jnp
from jax import lax
from jax.experimental import pallas as pl
from jax.experimental.pallas import tpu as pltpu
from jax.experimental.pallas import tpu_sc as plsc

D_H = 4096
NUM_ASPECTS = 6
BATCH = 4096

_NC = 2
_NS = 16
_NW = _NC * _NS
_BPW = BATCH // _NW
_C = 8
_NCHUNK = _BPW // _C
_CH_ELTS = _C * D_H


@functools.partial(
    pl.kernel,
    mesh=plsc.VectorSubcoreMesh(core_axis_name="c", subcore_axis_name="s"),
    out_type=jax.ShapeDtypeStruct((BATCH * D_H,), jnp.float32),
    scratch_types=[
        pltpu.VMEM((_BPW,), jnp.int32),
        pltpu.VMEM((NUM_ASPECTS * D_H,), jnp.float32),
        pltpu.VMEM((_CH_ELTS,), jnp.float32),
        pltpu.VMEM((_CH_ELTS,), jnp.float32),
        pltpu.SemaphoreType.DMA,
    ],
)
def _lookup(idx_hbm, table_hbm, out_hbm, idx_v, table_v, buf0, buf1, wsem):
    wid = lax.axis_index("s") * _NC + lax.axis_index("c")
    base = wid * _BPW
    pltpu.sync_copy(table_hbm, table_v)
    pltpu.sync_copy(idx_hbm.at[pl.ds(base, _BPW)], idx_v)

    bufs = (buf0, buf1)

    def step(p, carry):
        idx16 = idx_v[pl.ds(p * 16, 16)]
        for k in range(2):
            g = p * 2 + k
            buf = bufs[k]

            @pl.when(p >= 1)
            def _():
                pltpu.make_async_copy(
                    buf, out_hbm.at[pl.ds((base + (g - 2) * _C) * D_H,
                                          _CH_ELTS)], wsem).wait()

            for r in range(_C):
                off = idx16[k * _C + r] * D_H
                dst = r * D_H

                @plsc.parallel_loop(0, D_H, step=16, unroll=16)
                def _(c):
                    buf[pl.ds(dst + c, 16)] = table_v[pl.ds(off + c, 16)]

            pltpu.async_copy(
                buf, out_hbm.at[pl.ds((base + g * _C) * D_H, _CH_ELTS)], wsem)
        return carry

    lax.fori_loop(0, _NCHUNK // 2, step, 0)
    for k in range(2):
        g = _NCHUNK - 2 + k
        pltpu.make_async_copy(
            bufs[k], out_hbm.at[pl.ds((base + g * _C) * D_H, _CH_ELTS)],
            wsem).wait()


def kernel(aspect_idx, embed_weight):
    out = _lookup(aspect_idx.astype(jnp.int32), embed_weight.reshape(-1))
    return out.reshape(BATCH, D_H)


# speedup vs baseline: 2.1819x; 2.1819x over previous
import functools

import jax
import jax.numpy as jnp
from jax import lax
from jax.experimental import pallas as pl
from jax.experimental.pallas import tpu as pltpu
from jax.experimental.pallas import tpu_sc as plsc

D_H = 4096
NUM_ASPECTS = 6
BATCH = 4096

_NC = 2
_NS = 16
_NW = _NC * _NS
_BPW = BATCH // _NW
_L = 16
_NGRP = _BPW // _L


@functools.partial(
    pl.kernel,
    mesh=plsc.VectorSubcoreMesh(core_axis_name="c", subcore_axis_name="s"),
    out_type=jax.ShapeDtypeStruct((BATCH, D_H), jnp.float32),
    scratch_types=[
        pltpu.VMEM((_BPW,), jnp.int32),
        pltpu.VMEM((NUM_ASPECTS, D_H), jnp.float32),
        pltpu.SemaphoreType.DMA,
        pltpu.SemaphoreType.DMA,
    ],
)
def _lookup(idx_hbm, table_hbm, out_hbm, idx_v, table_v, sem0, sem1):
    sems = (sem0, sem1)
    wid = lax.axis_index("s") * _NC + lax.axis_index("c")
    base = wid * _BPW
    pltpu.sync_copy(table_hbm, table_v)
    pltpu.sync_copy(idx_hbm.at[pl.ds(base, _BPW)], idx_v)

    copies = []
    for g in range(_NGRP):
        idx16 = idx_v[pl.ds(g * _L, _L)]
        for j in range(_L):
            sj = idx16[j]
            row = base + g * _L + j
            copies.append(pltpu.make_async_copy(
                table_v.at[pl.ds(sj, 1)], out_hbm.at[pl.ds(row, 1)],
                sems[(g * _L + j) % 2]))
    for c in copies:
        c.start()
    for c in copies:
        c.wait()


def kernel(aspect_idx, embed_weight):
    return _lookup(aspect_idx.astype(jnp.int32), embed_weight)
